# baseline (speedup 1.0000x reference)
import functools

import jax
import jax.numpy as jnp
from jax import lax
from jax.experimental import pallas as pl
from jax.experimental.pallas import tpu as pltpu
from jax.experimental.pallas import tpu_sc as plsc

N = 10000
NP = 10240
E = 320000
EP = 327680
L = 8
D = 100
DP = 128
H = 100
B = 128
R = 512
GRID = NP // R
NW = 32
TROWS = NP // 16


ECH = EP // NW // 128
ECHS = 16
NCH = EP // 128
FASTC = 0
FCH = 160
SCH = 0
GCH = L * NP // NW // 128


def _emb_gather_body(emb_hbm, idx3_hbm, out_hbm, eidx2, rows0, rows1,
                     gs0, gs1):
    c = lax.axis_index("c")
    s = lax.axis_index("s")
    wid = s * 2 + c
    lrow = wid // 4
    r0 = (wid % 4) * (GCH * 128)
    pltpu.sync_copy(idx3_hbm.at[wid], eidx2)

    pltpu.async_copy(emb_hbm.at[eidx2.at[0]], rows0, gs0)

    def body(j, carry):
        c0 = 2 * j
        c1 = 2 * j + 1
        pltpu.async_copy(emb_hbm.at[eidx2.at[c1]], rows1, gs1)
        pltpu.make_async_copy(emb_hbm.at[eidx2.at[c0]], rows0, gs0).wait()
        pltpu.sync_copy(rows0, out_hbm.at[lrow, pl.ds(r0 + c0 * 128, 128)])

        @pl.when(j < GCH // 2 - 1)
        def _():
            pltpu.async_copy(emb_hbm.at[eidx2.at[c0 + 2]], rows0, gs0)

        pltpu.make_async_copy(emb_hbm.at[eidx2.at[c1]], rows1, gs1).wait()
        pltpu.sync_copy(rows1, out_hbm.at[lrow, pl.ds(r0 + c1 * 128, 128)])
        return carry

    lax.fori_loop(0, GCH // 2, body, 0)


def _degree_body(dst2_hbm, out_hbm, didx2, ones_v, buf_v, acc):
    c = lax.axis_index("c")
    s = lax.axis_index("s")
    wid = s * 2 + c

    for j in range(8):
        ones_v[pl.ds(j * 16, 16)] = jnp.ones((16,), jnp.float32)

    def zrow(i, carry):
        buf_v[pl.ds(i * 16, 16)] = jnp.zeros((16,), jnp.float32)
        return carry

    lax.fori_loop(0, TROWS // 16, zrow, 0)
    pltpu.sync_copy(buf_v, acc.at[pl.ds(s * TROWS, TROWS)])
    plsc.subcore_barrier()

    def outer(o, carry):
        row0 = wid * ECH + o * ECHS
        pltpu.sync_copy(dst2_hbm.at[pl.ds(row0, ECHS)], didx2)

        def body(i, carry2):
            pltpu.sync_copy(ones_v, acc.at[didx2.at[i]], add=True)
            return carry2

        lax.fori_loop(0, ECHS, body, 0)
        return carry

    lax.fori_loop(0, ECH // ECHS, outer, 0)
    plsc.subcore_barrier()
    pltpu.sync_copy(acc.at[pl.ds(s * TROWS, TROWS)], buf_v)
    pltpu.sync_copy(buf_v, out_hbm.at[c, pl.ds(s * TROWS, TROWS)])


def _edge_acc_body(xws_hbm, src2_hbm, dst2_hbm, out_hbm, sidx2, didx2,
                   rows0, rows1, acc, gs0, gs1):
    c = lax.axis_index("c")
    s = lax.axis_index("s")

    def zrow(i, carry):
        for j in range(DP // 16):
            rows0[i, pl.ds(j * 16, 16)] = jnp.zeros((16,), jnp.float32)
        return carry

    lax.fori_loop(0, 128, zrow, 0)

    def zcp(i, carry):
        pltpu.sync_copy(rows0, acc.at[pl.ds(s * TROWS + i * 128, 128)])
        return carry

    lax.fori_loop(0, TROWS // 128, zcp, 0)
    plsc.subcore_barrier()

    nseg = jnp.where(c == FASTC, FCH // ECHS, SCH // ECHS)
    cstart = jnp.where(c == FASTC, s * FCH, 16 * FCH + s * SCH)

    def outer(o, carry):
        row0 = cstart + o * ECHS
        pltpu.sync_copy(src2_hbm.at[pl.ds(row0, ECHS)], sidx2)
        pltpu.sync_copy(dst2_hbm.at[pl.ds(row0, ECHS)], didx2)
        pltpu.async_copy(xws_hbm.at[sidx2.at[0]], rows0, gs0)

        def body(j, carry2):
            c0 = 2 * j
            c1 = 2 * j + 1
            pltpu.async_copy(xws_hbm.at[sidx2.at[c1]], rows1, gs1)
            pltpu.make_async_copy(xws_hbm.at[sidx2.at[c0]], rows0, gs0).wait()
            pltpu.sync_copy(rows0, acc.at[didx2.at[c0]], add=True)

            @pl.when(j < ECHS // 2 - 1)
            def _():
                pltpu.async_copy(xws_hbm.at[sidx2.at[c0 + 2]], rows0, gs0)

            pltpu.make_async_copy(xws_hbm.at[sidx2.at[c1]], rows1, gs1).wait()
            pltpu.sync_copy(rows1, acc.at[didx2.at[c1]], add=True)
            return carry2

        lax.fori_loop(0, ECHS // 2, body, 0)
        return carry

    lax.fori_loop(0, nseg, outer, 0)
    plsc.subcore_barrier()

    def wb(i, carry):
        r0 = s * TROWS + i * 128
        pltpu.sync_copy(acc.at[pl.ds(r0, 128)], rows0)
        pltpu.sync_copy(rows0, out_hbm.at[c, pl.ds(r0, 128)])
        return carry

    lax.fori_loop(0, TROWS // 128, wb, 0)


_SC_CACHE = {}


def _sc_kernels():
    if "emb" not in _SC_CACHE:
        mesh = plsc.VectorSubcoreMesh(core_axis_name="c", subcore_axis_name="s")
        _SC_CACHE["emb"] = pl.kernel(
            _emb_gather_body,
            out_type=jax.ShapeDtypeStruct((L, NP, DP), jnp.float32),
            mesh=mesh,
            scratch_types=[
                pltpu.VMEM((GCH, 128), jnp.int32),
                pltpu.VMEM((128, DP), jnp.float32),
                pltpu.VMEM((128, DP), jnp.float32),
                pltpu.SemaphoreType.DMA,
                pltpu.SemaphoreType.DMA,
            ],
        )
        _SC_CACHE["deg"] = pl.kernel(
            _degree_body,
            out_type=jax.ShapeDtypeStruct((2, NP), jnp.float32),
            mesh=mesh,
            scratch_types=[
                pltpu.VMEM((ECHS, 128), jnp.int32),
                pltpu.VMEM((128,), jnp.float32),
                pltpu.VMEM((TROWS,), jnp.float32),
                pltpu.VMEM_SHARED((NP,), jnp.float32),
            ],
        )
        _SC_CACHE["edge"] = pl.kernel(
            _edge_acc_body,
            out_type=jax.ShapeDtypeStruct((2, NP, DP), jnp.float32),
            mesh=mesh,
            scratch_types=[
                pltpu.VMEM((ECHS, 128), jnp.int32),
                pltpu.VMEM((ECHS, 128), jnp.int32),
                pltpu.VMEM((128, DP), jnp.float32),
                pltpu.VMEM((128, DP), jnp.float32),
                pltpu.VMEM_SHARED((NP, DP), jnp.float32),
                pltpu.SemaphoreType.DMA,
                pltpu.SemaphoreType.DMA,
            ],
        )
    return _SC_CACHE["emb"], _SC_CACHE["deg"], _SC_CACHE["edge"]


VCHUNK = 10000


def _pad_body(emb_ref, out_ref):
    x = emb_ref[...]
    out_ref[...] = jnp.pad(x, ((0, 0), (0, DP - D)))


_pad_call = pl.pallas_call(
    _pad_body,
    grid=(100000 // VCHUNK,),
    in_specs=[pl.BlockSpec((VCHUNK, D), lambda i: (i, 0))],
    out_specs=pl.BlockSpec((VCHUNK, DP), lambda i: (i, 0)),
    out_shape=jax.ShapeDtypeStruct((100000, DP), jnp.float32),
)


def _sig(x):
    return jax.nn.sigmoid(x)


def _elu(x):
    return jnp.where(x > 0, x, jnp.exp(jnp.minimum(x, 0.0)) - 1.0)


def _cell(x, h, wr, wz, wn, ur, uz, un, br, bz, bin_, bhn):
    r = _sig(jnp.dot(x, wr) + jnp.dot(h, ur) + br)
    z = _sig(jnp.dot(x, wz) + jnp.dot(h, uz) + bz)
    n = jnp.tanh(jnp.dot(x, wn) + bin_ + r * (jnp.dot(h, un) + bhn))
    return (1.0 - z) * n + z * h


def _gru_body(xg_ref, h00_ref, h01_ref, wg_ref, bg_ref, w1_ref, degt_ref,
              hlast_ref, xw1s_ref):
    h0 = h00_ref[...]
    h1 = h01_ref[...]
    w = [wg_ref[k] for k in range(12)]
    b = [bg_ref[k:k + 1, :] for k in range(8)]
    for t in range(L):
        x = xg_ref[t]
        h0 = _cell(x, h0, w[0], w[1], w[2], w[3], w[4], w[5],
                   b[0], b[1], b[2], b[3])
        h1 = _cell(h0, h1, w[6], w[7], w[8], w[9], w[10], w[11],
                   b[4], b[5], b[6], b[7])
    hlast_ref[...] = h1
    dinv = lax.rsqrt(degt_ref[:, 0:1] + degt_ref[:, 1:2] + 1.0)
    rows = pl.program_id(0) * R + lax.broadcasted_iota(jnp.int32, (R, 1), 0)
    xw1s_ref[...] = jnp.where(rows < N, dinv * jnp.dot(h1, w1_ref[...]), 0.0)


def _tcb_body(acc_ref, xw1s_ref, degt_ref, x1r_ref, idx_ref, w2a_ref, w2b_ref,
              b1_ref, xw2s_ref, x2_ref):
    acc = acc_ref[0] + acc_ref[1]
    dinv = lax.rsqrt(degt_ref[:, 0:1] + degt_ref[:, 1:2] + 1.0)
    x2 = dinv * (acc + xw1s_ref[...]) + b1_ref[...]
    onehot = (idx_ref[...] ==
              lax.broadcasted_iota(jnp.int32, (R, 128), 1)).astype(jnp.float32)
    root = jnp.dot(onehot, x1r_ref[...])
    xw2 = jnp.dot(_elu(x2), w2a_ref[...]) + jnp.dot(_elu(root), w2b_ref[...])
    rows = pl.program_id(0) * R + lax.broadcasted_iota(jnp.int32, (R, 1), 0)
    xw2s_ref[...] = jnp.where(rows < N, dinv * xw2, 0.0)
    x2_ref[...] = x2


def _tcc_body(acc_ref, xw2s_ref, degt_ref, x2r_ref, idx_ref, b2_ref,
              oa_ref, ob_ref, oc_ref):
    i = pl.program_id(0)
    acc = acc_ref[0] + acc_ref[1]
    dinv = lax.rsqrt(degt_ref[:, 0:1] + degt_ref[:, 1:2] + 1.0)
    xelu = _elu(dinv * (acc + xw2s_ref[...]) + b2_ref[...])
    onehot = (idx_ref[...] ==
              lax.broadcasted_iota(jnp.int32, (R, 128), 1)).astype(jnp.float32)
    root2 = jnp.dot(onehot, x2r_ref[...])
    dn = (((0,), (0,)), ((), ()))
    sa = lax.dot_general(onehot, xelu, dn, preferred_element_type=jnp.float32)
    sb = lax.dot_general(onehot, root2, dn, preferred_element_type=jnp.float32)
    cnt = lax.dot_general(onehot, jnp.ones((R, 1), jnp.float32), dn,
                          preferred_element_type=jnp.float32)

    @pl.when(i == 0)
    def _():
        oa_ref[...] = sa
        ob_ref[...] = sb
        oc_ref[...] = cnt

    @pl.when(i > 0)
    def _():
        oa_ref[...] += sa
        ob_ref[...] += sb
        oc_ref[...] += cnt

    @pl.when(i == GRID - 1)
    def _():
        cfull = jnp.maximum(oc_ref[...], 1.0)
        oa_ref[...] = oa_ref[...] / cfull
        ob_ref[...] = ob_ref[...] / cfull


_gru_call = pl.pallas_call(
    _gru_body,
    grid=(GRID,),
    in_specs=[
        pl.BlockSpec((L, R, DP), lambda i: (0, i, 0)),
        pl.BlockSpec((R, DP), lambda i: (i, 0)),
        pl.BlockSpec((R, DP), lambda i: (i, 0)),
        pl.BlockSpec((12, DP, DP), lambda i: (0, 0, 0)),
        pl.BlockSpec((8, DP), lambda i: (0, 0)),
        pl.BlockSpec((DP, DP), lambda i: (0, 0)),
        pl.BlockSpec((R, 2), lambda i: (i, 0)),
    ],
    out_specs=[
        pl.BlockSpec((R, DP), lambda i: (i, 0)),
        pl.BlockSpec((R, DP), lambda i: (i, 0)),
    ],
    out_shape=[
        jax.ShapeDtypeStruct((NP, DP), jnp.float32),
        jax.ShapeDtypeStruct((NP, DP), jnp.float32),
    ],
)

_tcb_call = pl.pallas_call(
    _tcb_body,
    grid=(GRID,),
    in_specs=[
        pl.BlockSpec((2, R, DP), lambda i: (0, i, 0)),
        pl.BlockSpec((R, DP), lambda i: (i, 0)),
        pl.BlockSpec((R, 2), lambda i: (i, 0)),
        pl.BlockSpec((128, DP), lambda i: (0, 0)),
        pl.BlockSpec((R, 1), lambda i: (i, 0)),
        pl.BlockSpec((DP, DP), lambda i: (0, 0)),
        pl.BlockSpec((DP, DP), lambda i: (0, 0)),
        pl.BlockSpec((1, DP), lambda i: (0, 0)),
    ],
    out_specs=[
        pl.BlockSpec((R, DP), lambda i: (i, 0)),
        pl.BlockSpec((R, DP), lambda i: (i, 0)),
    ],
    out_shape=[
        jax.ShapeDtypeStruct((NP, DP), jnp.float32),
        jax.ShapeDtypeStruct((NP, DP), jnp.float32),
    ],
)

_tcc_call = pl.pallas_call(
    _tcc_body,
    grid=(GRID,),
    in_specs=[
        pl.BlockSpec((2, R, DP), lambda i: (0, i, 0)),
        pl.BlockSpec((R, DP), lambda i: (i, 0)),
        pl.BlockSpec((R, 2), lambda i: (i, 0)),
        pl.BlockSpec((128, DP), lambda i: (0, 0)),
        pl.BlockSpec((R, 1), lambda i: (i, 0)),
        pl.BlockSpec((1, DP), lambda i: (0, 0)),
    ],
    out_specs=[
        pl.BlockSpec((128, 128), lambda i: (0, 0)),
        pl.BlockSpec((128, 128), lambda i: (0, 0)),
        pl.BlockSpec((128, 1), lambda i: (0, 0)),
    ],
    out_shape=[
        jax.ShapeDtypeStruct((128, 128), jnp.float32),
        jax.ShapeDtypeStruct((128, 128), jnp.float32),
        jax.ShapeDtypeStruct((128, 1), jnp.float32),
    ],
)


def _padw(w):
    return jnp.pad(w, ((0, DP - w.shape[0]), (0, DP - w.shape[1])))


def kernel(merged_tree_feature, merged_tree_edge_index, indices,
           emb, Wih0, Whh0, bih0, bhh0, Wih1, Whh1, bih1, bhh1,
           h0, W1, b1, W2, b2):
    f32 = jnp.float32
    feat3 = jnp.pad(merged_tree_feature.T.astype(jnp.int32),
                    ((0, 0), (0, NP - N))).reshape(NW, GCH, 128)
    src = merged_tree_edge_index[0].astype(jnp.int32)
    dst = merged_tree_edge_index[1].astype(jnp.int32)
    src2 = jnp.pad(src, (0, EP - E),
                   constant_values=NP - 1).reshape(NCH, 128)
    dst2 = jnp.pad(dst, (0, EP - E),
                   constant_values=NP - 1).reshape(NCH, 128)
    idx_p = jnp.pad(indices.astype(jnp.int32), (0, NP - N),
                    constant_values=-1).reshape(NP, 1)
    h00 = jnp.pad(h0[0], ((0, NP - N), (0, DP - H)))
    h01 = jnp.pad(h0[1], ((0, NP - N), (0, DP - H)))

    def gates(Wih, Whh):
        wr, wz, wn = Wih[0:H].T, Wih[H:2 * H].T, Wih[2 * H:3 * H].T
        ur, uz, un = Whh[0:H].T, Whh[H:2 * H].T, Whh[2 * H:3 * H].T
        return [_padw(m) for m in (wr, wz, wn, ur, uz, un)]

    wg = jnp.stack(gates(Wih0, Whh0) + gates(Wih1, Whh1))

    def bvec(v):
        return jnp.pad(v, (0, DP - H))

    bg = jnp.stack([
        bvec(bih0[0:H] + bhh0[0:H]), bvec(bih0[H:2 * H] + bhh0[H:2 * H]),
        bvec(bih0[2 * H:]), bvec(bhh0[2 * H:]),
        bvec(bih1[0:H] + bhh1[0:H]), bvec(bih1[H:2 * H] + bhh1[H:2 * H]),
        bvec(bih1[2 * H:]), bvec(bhh1[2 * H:]),
    ])
    w1_p = _padw(W1)
    w2a = _padw(W2[0:H])
    w2b = _padw(W2[H:2 * H])
    b1_p = bvec(b1).reshape(1, DP)
    b2_p = bvec(b2).reshape(1, DP)

    _emb_gather, _degree, _edge_acc = _sc_kernels()
    emb_p = _pad_call(emb)
    xg = _emb_gather(emb_p, feat3)
    degp = _degree(dst2)
    degt = degp.T

    h_last, xw1s = _gru_call(xg, h00, h01, wg, bg, w1_p, degt)

    acc1 = _edge_acc(xw1s, src2, dst2)

    xw2s, x2 = _tcb_call(acc1, xw1s, degt, h_last, idx_p, w2a, w2b, b1_p)

    acc2 = _edge_acc(xw2s, src2, dst2)

    oa, ob, _ = _tcc_call(acc2, xw2s, degt, x2, idx_p, b2_p)
    return jnp.concatenate([oa[:, 0:H], ob[:, 0:H]], axis=1).astype(f32)


# speedup vs baseline: 1.2015x; 1.2015x over previous
import functools

import jax
import jax.numpy as jnp
from jax import lax
from jax.experimental import pallas as pl
from jax.experimental.pallas import tpu as pltpu
from jax.experimental.pallas import tpu_sc as plsc

N = 10000
NP = 10240
E = 320000
EP = 327680
L = 8
D = 100
DP = 128
H = 100
B = 128
R = 512
GRID = NP // R
NW = 32
TROWS = NP // 16


ECH = EP // NW // 128
ECHS = 16
NCH = EP // 128
FASTC = 0
FCH = 144
SCH = 16
NGCH = L * NP // 128
GCHS = 8
GFCH = 32
GSCH = 8


def _emb_gather_body(emb_hbm, idx2_hbm, out_hbm, eidx2, rows0, rows1,
                     gs0, gs1):
    c = lax.axis_index("c")
    s = lax.axis_index("s")
    nseg = jnp.where(c == FASTC, GFCH // GCHS, GSCH // GCHS)
    cstart = jnp.where(c == FASTC, s * GFCH, 16 * GFCH + s * GSCH)

    def outer(o, carry):
        ch0 = cstart + o * GCHS
        pltpu.sync_copy(idx2_hbm.at[pl.ds(ch0, GCHS)], eidx2)
        pltpu.async_copy(emb_hbm.at[eidx2.at[0]], rows0, gs0)

        def body(j, carry2):
            c0 = 2 * j
            c1 = 2 * j + 1
            pltpu.async_copy(emb_hbm.at[eidx2.at[c1]], rows1, gs1)
            pltpu.make_async_copy(emb_hbm.at[eidx2.at[c0]], rows0, gs0).wait()
            pltpu.sync_copy(rows0, out_hbm.at[pl.ds((ch0 + c0) * 128, 128)])

            @pl.when(j < GCHS // 2 - 1)
            def _():
                pltpu.async_copy(emb_hbm.at[eidx2.at[c0 + 2]], rows0, gs0)

            pltpu.make_async_copy(emb_hbm.at[eidx2.at[c1]], rows1, gs1).wait()
            pltpu.sync_copy(rows1, out_hbm.at[pl.ds((ch0 + c1) * 128, 128)])
            return carry2

        lax.fori_loop(0, GCHS // 2, body, 0)
        return carry

    lax.fori_loop(0, nseg, outer, 0)


def _degree_body(dst2_hbm, out_hbm, didx2, ones_v, buf_v, acc):
    c = lax.axis_index("c")
    s = lax.axis_index("s")
    wid = s * 2 + c

    for j in range(8):
        ones_v[pl.ds(j * 16, 16)] = jnp.ones((16,), jnp.float32)

    def zrow(i, carry):
        buf_v[pl.ds(i * 16, 16)] = jnp.zeros((16,), jnp.float32)
        return carry

    lax.fori_loop(0, TROWS // 16, zrow, 0)
    pltpu.sync_copy(buf_v, acc.at[pl.ds(s * TROWS, TROWS)])
    plsc.subcore_barrier()

    def outer(o, carry):
        row0 = wid * ECH + o * ECHS
        pltpu.sync_copy(dst2_hbm.at[pl.ds(row0, ECHS)], didx2)

        def body(i, carry2):
            pltpu.sync_copy(ones_v, acc.at[didx2.at[i]], add=True)
            return carry2

        lax.fori_loop(0, ECHS, body, 0)
        return carry

    lax.fori_loop(0, ECH // ECHS, outer, 0)
    plsc.subcore_barrier()
    pltpu.sync_copy(acc.at[pl.ds(s * TROWS, TROWS)], buf_v)
    pltpu.sync_copy(buf_v, out_hbm.at[c, pl.ds(s * TROWS, TROWS)])


def _edge_acc_body(xws_hbm, src2_hbm, dst2_hbm, out_hbm, sidx2, didx2,
                   rows0, rows1, acc, gs0, gs1):
    c = lax.axis_index("c")
    s = lax.axis_index("s")

    def zrow(i, carry):
        for j in range(DP // 16):
            rows0[i, pl.ds(j * 16, 16)] = jnp.zeros((16,), jnp.float32)
        return carry

    lax.fori_loop(0, 128, zrow, 0)

    def zcp(i, carry):
        pltpu.sync_copy(rows0, acc.at[pl.ds(s * TROWS + i * 128, 128)])
        return carry

    lax.fori_loop(0, TROWS // 128, zcp, 0)
    plsc.subcore_barrier()

    nseg = jnp.where(c == FASTC, FCH // ECHS, SCH // ECHS)
    cstart = jnp.where(c == FASTC, s * FCH, 16 * FCH + s * SCH)

    def outer(o, carry):
        row0 = cstart + o * ECHS
        pltpu.sync_copy(src2_hbm.at[pl.ds(row0, ECHS)], sidx2)
        pltpu.sync_copy(dst2_hbm.at[pl.ds(row0, ECHS)], didx2)
        pltpu.async_copy(xws_hbm.at[sidx2.at[0]], rows0, gs0)

        def body(j, carry2):
            c0 = 2 * j
            c1 = 2 * j + 1
            pltpu.async_copy(xws_hbm.at[sidx2.at[c1]], rows1, gs1)
            pltpu.make_async_copy(xws_hbm.at[sidx2.at[c0]], rows0, gs0).wait()
            pltpu.sync_copy(rows0, acc.at[didx2.at[c0]], add=True)

            @pl.when(j < ECHS // 2 - 1)
            def _():
                pltpu.async_copy(xws_hbm.at[sidx2.at[c0 + 2]], rows0, gs0)

            pltpu.make_async_copy(xws_hbm.at[sidx2.at[c1]], rows1, gs1).wait()
            pltpu.sync_copy(rows1, acc.at[didx2.at[c1]], add=True)
            return carry2

        lax.fori_loop(0, ECHS // 2, body, 0)
        return carry

    lax.fori_loop(0, nseg, outer, 0)
    plsc.subcore_barrier()

    def wb(i, carry):
        r0 = s * TROWS + i * 128
        pltpu.sync_copy(acc.at[pl.ds(r0, 128)], rows0)
        pltpu.sync_copy(rows0, out_hbm.at[c, pl.ds(r0, 128)])
        return carry

    lax.fori_loop(0, TROWS // 128, wb, 0)


_SC_CACHE = {}


def _sc_kernels():
    if "emb" not in _SC_CACHE:
        mesh = plsc.VectorSubcoreMesh(core_axis_name="c", subcore_axis_name="s")
        _SC_CACHE["emb"] = pl.kernel(
            _emb_gather_body,
            out_type=jax.ShapeDtypeStruct((L * NP, DP), jnp.float32),
            mesh=mesh,
            scratch_types=[
                pltpu.VMEM((GCHS, 128), jnp.int32),
                pltpu.VMEM((128, DP), jnp.float32),
                pltpu.VMEM((128, DP), jnp.float32),
                pltpu.SemaphoreType.DMA,
                pltpu.SemaphoreType.DMA,
            ],
        )
        _SC_CACHE["deg"] = pl.kernel(
            _degree_body,
            out_type=jax.ShapeDtypeStruct((2, NP), jnp.float32),
            mesh=mesh,
            scratch_types=[
                pltpu.VMEM((ECHS, 128), jnp.int32),
                pltpu.VMEM((128,), jnp.float32),
                pltpu.VMEM((TROWS,), jnp.float32),
                pltpu.VMEM_SHARED((NP,), jnp.float32),
            ],
        )
        _SC_CACHE["edge"] = pl.kernel(
            _edge_acc_body,
            out_type=jax.ShapeDtypeStruct((2, NP, DP), jnp.float32),
            mesh=mesh,
            scratch_types=[
                pltpu.VMEM((ECHS, 128), jnp.int32),
                pltpu.VMEM((ECHS, 128), jnp.int32),
                pltpu.VMEM((128, DP), jnp.float32),
                pltpu.VMEM((128, DP), jnp.float32),
                pltpu.VMEM_SHARED((NP, DP), jnp.float32),
                pltpu.SemaphoreType.DMA,
                pltpu.SemaphoreType.DMA,
            ],
        )
    return _SC_CACHE["emb"], _SC_CACHE["deg"], _SC_CACHE["edge"]


VCHUNK = 10000


def _pad_body(emb_ref, out_ref):
    x = emb_ref[...]
    out_ref[...] = jnp.pad(x, ((0, 0), (0, DP - D)))


_pad_call = pl.pallas_call(
    _pad_body,
    grid=(100000 // VCHUNK,),
    in_specs=[pl.BlockSpec((VCHUNK, D), lambda i: (i, 0))],
    out_specs=pl.BlockSpec((VCHUNK, DP), lambda i: (i, 0)),
    out_shape=jax.ShapeDtypeStruct((100000, DP), jnp.float32),
)


def _sig(x):
    return jax.nn.sigmoid(x)


def _elu(x):
    return jnp.where(x > 0, x, jnp.exp(jnp.minimum(x, 0.0)) - 1.0)


def _cell(x, h, wr, wz, wn, ur, uz, un, br, bz, bin_, bhn):
    r = _sig(jnp.dot(x, wr) + jnp.dot(h, ur) + br)
    z = _sig(jnp.dot(x, wz) + jnp.dot(h, uz) + bz)
    n = jnp.tanh(jnp.dot(x, wn) + bin_ + r * (jnp.dot(h, un) + bhn))
    return (1.0 - z) * n + z * h


def _gru_body(xg_ref, h00_ref, h01_ref, wg_ref, bg_ref, w1_ref, degt_ref,
              hlast_ref, xw1s_ref):
    zpad = jnp.zeros((R, DP - H), jnp.float32)
    h0 = jnp.concatenate([h00_ref[...], zpad], axis=1)
    h1 = jnp.concatenate([h01_ref[...], zpad], axis=1)
    w = [wg_ref[k] for k in range(12)]
    b = [bg_ref[k:k + 1, :] for k in range(8)]
    for t in range(L):
        x = xg_ref[t]
        h0 = _cell(x, h0, w[0], w[1], w[2], w[3], w[4], w[5],
                   b[0], b[1], b[2], b[3])
        h1 = _cell(h0, h1, w[6], w[7], w[8], w[9], w[10], w[11],
                   b[4], b[5], b[6], b[7])
    hlast_ref[...] = h1
    dinv = lax.rsqrt(degt_ref[:, 0:1] + degt_ref[:, 1:2] + 1.0)
    rows = pl.program_id(0) * R + lax.broadcasted_iota(jnp.int32, (R, 1), 0)
    xw1s_ref[...] = jnp.where(rows < N, dinv * jnp.dot(h1, w1_ref[...]), 0.0)


def _tcb_body(acc_ref, xw1s_ref, degt_ref, x1r_ref, idx_ref, w2a_ref, w2b_ref,
              b1_ref, xw2s_ref, x2_ref):
    acc = acc_ref[0] + acc_ref[1]
    dinv = lax.rsqrt(degt_ref[:, 0:1] + degt_ref[:, 1:2] + 1.0)
    x2 = dinv * (acc + xw1s_ref[...]) + b1_ref[...]
    onehot = (idx_ref[...] ==
              lax.broadcasted_iota(jnp.int32, (R, 128), 1)).astype(jnp.float32)
    root = jnp.dot(onehot, x1r_ref[...])
    xw2 = jnp.dot(_elu(x2), w2a_ref[...]) + jnp.dot(_elu(root), w2b_ref[...])
    rows = pl.program_id(0) * R + lax.broadcasted_iota(jnp.int32, (R, 1), 0)
    xw2s_ref[...] = jnp.where(rows < N, dinv * xw2, 0.0)
    x2_ref[...] = x2


def _tcc_body(acc_ref, xw2s_ref, degt_ref, x2r_ref, idx_ref, b2_ref,
              oa_ref, ob_ref, oc_ref):
    i = pl.program_id(0)
    acc = acc_ref[0] + acc_ref[1]
    dinv = lax.rsqrt(degt_ref[:, 0:1] + degt_ref[:, 1:2] + 1.0)
    xelu = _elu(dinv * (acc + xw2s_ref[...]) + b2_ref[...])
    onehot = (idx_ref[...] ==
              lax.broadcasted_iota(jnp.int32, (R, 128), 1)).astype(jnp.float32)
    root2 = jnp.dot(onehot, x2r_ref[...])
    dn = (((0,), (0,)), ((), ()))
    sa = lax.dot_general(onehot, xelu, dn, preferred_element_type=jnp.float32)
    sb = lax.dot_general(onehot, root2, dn, preferred_element_type=jnp.float32)
    cnt = lax.dot_general(onehot, jnp.ones((R, 1), jnp.float32), dn,
                          preferred_element_type=jnp.float32)

    @pl.when(i == 0)
    def _():
        oa_ref[...] = sa
        ob_ref[...] = sb
        oc_ref[...] = cnt

    @pl.when(i > 0)
    def _():
        oa_ref[...] += sa
        ob_ref[...] += sb
        oc_ref[...] += cnt

    @pl.when(i == GRID - 1)
    def _():
        cfull = jnp.maximum(oc_ref[...], 1.0)
        oa_ref[...] = oa_ref[...] / cfull
        ob_ref[...] = ob_ref[...] / cfull


_gru_call = pl.pallas_call(
    _gru_body,
    grid=(GRID,),
    in_specs=[
        pl.BlockSpec((L, R, DP), lambda i: (0, i, 0)),
        pl.BlockSpec((R, H), lambda i: (i, 0)),
        pl.BlockSpec((R, H), lambda i: (i, 0)),
        pl.BlockSpec((12, DP, DP), lambda i: (0, 0, 0)),
        pl.BlockSpec((8, DP), lambda i: (0, 0)),
        pl.BlockSpec((DP, DP), lambda i: (0, 0)),
        pl.BlockSpec((R, 2), lambda i: (i, 0)),
    ],
    out_specs=[
        pl.BlockSpec((R, DP), lambda i: (i, 0)),
        pl.BlockSpec((R, DP), lambda i: (i, 0)),
    ],
    out_shape=[
        jax.ShapeDtypeStruct((NP, DP), jnp.float32),
        jax.ShapeDtypeStruct((NP, DP), jnp.float32),
    ],
)

_tcb_call = pl.pallas_call(
    _tcb_body,
    grid=(GRID,),
    in_specs=[
        pl.BlockSpec((2, R, DP), lambda i: (0, i, 0)),
        pl.BlockSpec((R, DP), lambda i: (i, 0)),
        pl.BlockSpec((R, 2), lambda i: (i, 0)),
        pl.BlockSpec((128, DP), lambda i: (0, 0)),
        pl.BlockSpec((R, 1), lambda i: (i, 0)),
        pl.BlockSpec((DP, DP), lambda i: (0, 0)),
        pl.BlockSpec((DP, DP), lambda i: (0, 0)),
        pl.BlockSpec((1, DP), lambda i: (0, 0)),
    ],
    out_specs=[
        pl.BlockSpec((R, DP), lambda i: (i, 0)),
        pl.BlockSpec((R, DP), lambda i: (i, 0)),
    ],
    out_shape=[
        jax.ShapeDtypeStruct((NP, DP), jnp.float32),
        jax.ShapeDtypeStruct((NP, DP), jnp.float32),
    ],
)

_tcc_call = pl.pallas_call(
    _tcc_body,
    grid=(GRID,),
    in_specs=[
        pl.BlockSpec((2, R, DP), lambda i: (0, i, 0)),
        pl.BlockSpec((R, DP), lambda i: (i, 0)),
        pl.BlockSpec((R, 2), lambda i: (i, 0)),
        pl.BlockSpec((128, DP), lambda i: (0, 0)),
        pl.BlockSpec((R, 1), lambda i: (i, 0)),
        pl.BlockSpec((1, DP), lambda i: (0, 0)),
    ],
    out_specs=[
        pl.BlockSpec((128, 128), lambda i: (0, 0)),
        pl.BlockSpec((128, 128), lambda i: (0, 0)),
        pl.BlockSpec((128, 1), lambda i: (0, 0)),
    ],
    out_shape=[
        jax.ShapeDtypeStruct((128, 128), jnp.float32),
        jax.ShapeDtypeStruct((128, 128), jnp.float32),
        jax.ShapeDtypeStruct((128, 1), jnp.float32),
    ],
)


def _padw(w):
    return jnp.pad(w, ((0, DP - w.shape[0]), (0, DP - w.shape[1])))


def kernel(merged_tree_feature, merged_tree_edge_index, indices,
           emb, Wih0, Whh0, bih0, bhh0, Wih1, Whh1, bih1, bhh1,
           h0, W1, b1, W2, b2):
    f32 = jnp.float32
    feat2 = jnp.pad(merged_tree_feature.T.astype(jnp.int32),
                    ((0, 0), (0, NP - N))).reshape(NGCH, 128)
    src = merged_tree_edge_index[0].astype(jnp.int32)
    dst = merged_tree_edge_index[1].astype(jnp.int32)
    src2 = jnp.pad(src, (0, EP - E),
                   constant_values=NP - 1).reshape(NCH, 128)
    dst2 = jnp.pad(dst, (0, EP - E),
                   constant_values=NP - 1).reshape(NCH, 128)
    idx_p = jnp.pad(indices.astype(jnp.int32), (0, NP - N),
                    constant_values=-1).reshape(NP, 1)

    def gates(Wih, Whh):
        wr, wz, wn = Wih[0:H].T, Wih[H:2 * H].T, Wih[2 * H:3 * H].T
        ur, uz, un = Whh[0:H].T, Whh[H:2 * H].T, Whh[2 * H:3 * H].T
        return [_padw(m) for m in (wr, wz, wn, ur, uz, un)]

    wg = jnp.stack(gates(Wih0, Whh0) + gates(Wih1, Whh1))

    def bvec(v):
        return jnp.pad(v, (0, DP - H))

    bg = jnp.stack([
        bvec(bih0[0:H] + bhh0[0:H]), bvec(bih0[H:2 * H] + bhh0[H:2 * H]),
        bvec(bih0[2 * H:]), bvec(bhh0[2 * H:]),
        bvec(bih1[0:H] + bhh1[0:H]), bvec(bih1[H:2 * H] + bhh1[H:2 * H]),
        bvec(bih1[2 * H:]), bvec(bhh1[2 * H:]),
    ])
    w1_p = _padw(W1)
    w2a = _padw(W2[0:H])
    w2b = _padw(W2[H:2 * H])
    b1_p = bvec(b1).reshape(1, DP)
    b2_p = bvec(b2).reshape(1, DP)

    _emb_gather, _degree, _edge_acc = _sc_kernels()
    emb_p = _pad_call(emb)
    xg = _emb_gather(emb_p, feat2).reshape(L, NP, DP)
    degp = _degree(dst2)
    degt = degp.T

    h_last, xw1s = _gru_call(xg, h0[0], h0[1], wg, bg, w1_p, degt)

    acc1 = _edge_acc(xw1s, src2, dst2)

    xw2s, x2 = _tcb_call(acc1, xw1s, degt, h_last, idx_p, w2a, w2b, b1_p)

    acc2 = _edge_acc(xw2s, src2, dst2)

    oa, ob, _ = _tcc_call(acc2, xw2s, degt, x2, idx_p, b2_p)
    return jnp.concatenate([oa[:, 0:H], ob[:, 0:H]], axis=1).astype(f32)


# speedup vs baseline: 1.2446x; 1.0359x over previous
import functools

import jax
import jax.numpy as jnp
from jax import lax
from jax.experimental import pallas as pl
from jax.experimental.pallas import tpu as pltpu
from jax.experimental.pallas import tpu_sc as plsc

N = 10000
NP = 10240
E = 320000
EP = 327680
L = 8
D = 100
DP = 128
H = 100
B = 128
R = 512
GRID = NP // R
NW = 32
TROWS = NP // 16


ECH = EP // NW // 128
ECHS = 16
NCH = EP // 128
FASTC = 0
FCH = 144
SCH = 16
GCH = L * NP // NW // 128


def _emb_gather_body(emb_hbm, idx3_hbm, out_hbm, eidx2, rows0, rows1,
                     gs0, gs1):
    c = lax.axis_index("c")
    s = lax.axis_index("s")
    wid = s * 2 + c
    lrow = wid // 4
    r0 = (wid % 4) * (GCH * 128)
    pltpu.sync_copy(idx3_hbm.at[wid], eidx2)

    pltpu.async_copy(emb_hbm.at[eidx2.at[0]], rows0, gs0)

    def body(j, carry):
        c0 = 2 * j
        c1 = 2 * j + 1
        pltpu.async_copy(emb_hbm.at[eidx2.at[c1]], rows1, gs1)
        pltpu.make_async_copy(emb_hbm.at[eidx2.at[c0]], rows0, gs0).wait()
        pltpu.sync_copy(rows0, out_hbm.at[lrow, pl.ds(r0 + c0 * 128, 128)])

        @pl.when(j < GCH // 2 - 1)
        def _():
            pltpu.async_copy(emb_hbm.at[eidx2.at[c0 + 2]], rows0, gs0)

        pltpu.make_async_copy(emb_hbm.at[eidx2.at[c1]], rows1, gs1).wait()
        pltpu.sync_copy(rows1, out_hbm.at[lrow, pl.ds(r0 + c1 * 128, 128)])
        return carry

    lax.fori_loop(0, GCH // 2, body, 0)


def _degree_body(dst2_hbm, out_hbm, didx2, ones_v, buf_v, acc):
    c = lax.axis_index("c")
    s = lax.axis_index("s")
    wid = s * 2 + c

    for j in range(8):
        ones_v[pl.ds(j * 16, 16)] = jnp.ones((16,), jnp.float32)

    def zrow(i, carry):
        buf_v[pl.ds(i * 16, 16)] = jnp.zeros((16,), jnp.float32)
        return carry

    lax.fori_loop(0, TROWS // 16, zrow, 0)
    pltpu.sync_copy(buf_v, acc.at[pl.ds(s * TROWS, TROWS)])
    plsc.subcore_barrier()

    def outer(o, carry):
        row0 = wid * ECH + o * ECHS
        pltpu.sync_copy(dst2_hbm.at[pl.ds(row0, ECHS)], didx2)

        def body(i, carry2):
            pltpu.sync_copy(ones_v, acc.at[didx2.at[i]], add=True)
            return carry2

        lax.fori_loop(0, ECHS, body, 0)
        return carry

    lax.fori_loop(0, ECH // ECHS, outer, 0)
    plsc.subcore_barrier()
    pltpu.sync_copy(acc.at[pl.ds(s * TROWS, TROWS)], buf_v)
    pltpu.sync_copy(buf_v, out_hbm.at[c, pl.ds(s * TROWS, TROWS)])


def _edge_acc_body(xws_hbm, src2_hbm, dst2_hbm, out_hbm, sidx2, didx2,
                   rows0, rows1, acc, gs0, gs1):
    c = lax.axis_index("c")
    s = lax.axis_index("s")

    def zrow(i, carry):
        for j in range(DP // 16):
            rows0[i, pl.ds(j * 16, 16)] = jnp.zeros((16,), jnp.float32)
        return carry

    lax.fori_loop(0, 128, zrow, 0)

    def zcp(i, carry):
        pltpu.sync_copy(rows0, acc.at[pl.ds(s * TROWS + i * 128, 128)])
        return carry

    lax.fori_loop(0, TROWS // 128, zcp, 0)
    plsc.subcore_barrier()

    nseg = jnp.where(c == FASTC, FCH // ECHS, SCH // ECHS)
    cstart = jnp.where(c == FASTC, s * FCH, 16 * FCH + s * SCH)

    def outer(o, carry):
        row0 = cstart + o * ECHS
        pltpu.sync_copy(src2_hbm.at[pl.ds(row0, ECHS)], sidx2)
        pltpu.sync_copy(dst2_hbm.at[pl.ds(row0, ECHS)], didx2)
        pltpu.async_copy(xws_hbm.at[sidx2.at[0]], rows0, gs0)

        def body(j, carry2):
            c0 = 2 * j
            c1 = 2 * j + 1
            pltpu.async_copy(xws_hbm.at[sidx2.at[c1]], rows1, gs1)
            pltpu.make_async_copy(xws_hbm.at[sidx2.at[c0]], rows0, gs0).wait()
            pltpu.sync_copy(rows0, acc.at[didx2.at[c0]], add=True)

            @pl.when(j < ECHS // 2 - 1)
            def _():
                pltpu.async_copy(xws_hbm.at[sidx2.at[c0 + 2]], rows0, gs0)

            pltpu.make_async_copy(xws_hbm.at[sidx2.at[c1]], rows1, gs1).wait()
            pltpu.sync_copy(rows1, acc.at[didx2.at[c1]], add=True)
            return carry2

        lax.fori_loop(0, ECHS // 2, body, 0)
        return carry

    lax.fori_loop(0, nseg, outer, 0)
    plsc.subcore_barrier()

    def wb(i, carry):
        r0 = s * TROWS + i * 128
        pltpu.sync_copy(acc.at[pl.ds(r0, 128)], rows0)
        pltpu.sync_copy(rows0, out_hbm.at[c, pl.ds(r0, 128)])
        return carry

    lax.fori_loop(0, TROWS // 128, wb, 0)


_SC_CACHE = {}


def _sc_kernels():
    if "emb" not in _SC_CACHE:
        mesh = plsc.VectorSubcoreMesh(core_axis_name="c", subcore_axis_name="s")
        _SC_CACHE["emb"] = pl.kernel(
            _emb_gather_body,
            out_type=jax.ShapeDtypeStruct((L, NP, DP), jnp.float32),
            mesh=mesh,
            scratch_types=[
                pltpu.VMEM((GCH, 128), jnp.int32),
                pltpu.VMEM((128, DP), jnp.float32),
                pltpu.VMEM((128, DP), jnp.float32),
                pltpu.SemaphoreType.DMA,
                pltpu.SemaphoreType.DMA,
            ],
        )
        _SC_CACHE["deg"] = pl.kernel(
            _degree_body,
            out_type=jax.ShapeDtypeStruct((2, NP), jnp.float32),
            mesh=mesh,
            scratch_types=[
                pltpu.VMEM((ECHS, 128), jnp.int32),
                pltpu.VMEM((128,), jnp.float32),
                pltpu.VMEM((TROWS,), jnp.float32),
                pltpu.VMEM_SHARED((NP,), jnp.float32),
            ],
        )
        _SC_CACHE["edge"] = pl.kernel(
            _edge_acc_body,
            out_type=jax.ShapeDtypeStruct((2, NP, DP), jnp.float32),
            mesh=mesh,
            scratch_types=[
                pltpu.VMEM((ECHS, 128), jnp.int32),
                pltpu.VMEM((ECHS, 128), jnp.int32),
                pltpu.VMEM((128, DP), jnp.float32),
                pltpu.VMEM((128, DP), jnp.float32),
                pltpu.VMEM_SHARED((NP, DP), jnp.float32),
                pltpu.SemaphoreType.DMA,
                pltpu.SemaphoreType.DMA,
            ],
        )
    return _SC_CACHE["emb"], _SC_CACHE["deg"], _SC_CACHE["edge"]


VCHUNK = 10000


def _pad_body(emb_ref, out_ref):
    x = emb_ref[...]
    out_ref[...] = jnp.pad(x, ((0, 0), (0, DP - D)))


_pad_call = pl.pallas_call(
    _pad_body,
    grid=(100000 // VCHUNK,),
    in_specs=[pl.BlockSpec((VCHUNK, D), lambda i: (i, 0))],
    out_specs=pl.BlockSpec((VCHUNK, DP), lambda i: (i, 0)),
    out_shape=jax.ShapeDtypeStruct((100000, DP), jnp.float32),
)


def _sig(x):
    return jax.nn.sigmoid(x)


def _elu(x):
    return jnp.where(x > 0, x, jnp.exp(jnp.minimum(x, 0.0)) - 1.0)


def _dot(a, b):
    return jnp.dot(a, b, preferred_element_type=jnp.float32)


def _cell(x, h, wr, wz, wn, ur, uz, un, br, bz, bin_, bhn):
    hb = h.astype(jnp.bfloat16)
    r = _sig(_dot(x, wr) + _dot(hb, ur) + br)
    z = _sig(_dot(x, wz) + _dot(hb, uz) + bz)
    n = jnp.tanh(_dot(x, wn) + bin_ + r * (_dot(hb, un) + bhn))
    return (1.0 - z) * n + z * h


def _gru_body(xg_ref, h00_ref, h01_ref, wg_ref, bg_ref, w1_ref, degt_ref,
              hlast_ref, xw1s_ref):
    h0 = h00_ref[...]
    h1 = h01_ref[...]
    w = [wg_ref[k].astype(jnp.bfloat16) for k in range(12)]
    b = [bg_ref[k:k + 1, :] for k in range(8)]
    for t in range(L):
        x = xg_ref[t].astype(jnp.bfloat16)
        h0 = _cell(x, h0, w[0], w[1], w[2], w[3], w[4], w[5],
                   b[0], b[1], b[2], b[3])
        h1 = _cell(h0.astype(jnp.bfloat16), h1, w[6], w[7], w[8], w[9],
                   w[10], w[11], b[4], b[5], b[6], b[7])
    hlast_ref[...] = h1
    dinv = lax.rsqrt(degt_ref[:, 0:1] + degt_ref[:, 1:2] + 1.0)
    rows = pl.program_id(0) * R + lax.broadcasted_iota(jnp.int32, (R, 1), 0)
    xw1s_ref[...] = jnp.where(rows < N, dinv * jnp.dot(h1, w1_ref[...]), 0.0)


def _tcb_body(acc_ref, xw1s_ref, degt_ref, x1r_ref, idx_ref, w2a_ref, w2b_ref,
              b1_ref, xw2s_ref, x2_ref):
    acc = acc_ref[0] + acc_ref[1]
    dinv = lax.rsqrt(degt_ref[:, 0:1] + degt_ref[:, 1:2] + 1.0)
    x2 = dinv * (acc + xw1s_ref[...]) + b1_ref[...]
    onehot = (idx_ref[...] ==
              lax.broadcasted_iota(jnp.int32, (R, 128), 1)).astype(jnp.float32)
    root = jnp.dot(onehot, x1r_ref[...])
    xw2 = jnp.dot(_elu(x2), w2a_ref[...]) + jnp.dot(_elu(root), w2b_ref[...])
    rows = pl.program_id(0) * R + lax.broadcasted_iota(jnp.int32, (R, 1), 0)
    xw2s_ref[...] = jnp.where(rows < N, dinv * xw2, 0.0)
    x2_ref[...] = x2


def _tcc_body(acc_ref, xw2s_ref, degt_ref, x2r_ref, idx_ref, b2_ref,
              oa_ref, ob_ref, oc_ref):
    i = pl.program_id(0)
    acc = acc_ref[0] + acc_ref[1]
    dinv = lax.rsqrt(degt_ref[:, 0:1] + degt_ref[:, 1:2] + 1.0)
    xelu = _elu(dinv * (acc + xw2s_ref[...]) + b2_ref[...])
    onehot = (idx_ref[...] ==
              lax.broadcasted_iota(jnp.int32, (R, 128), 1)).astype(jnp.float32)
    root2 = jnp.dot(onehot, x2r_ref[...])
    dn = (((0,), (0,)), ((), ()))
    sa = lax.dot_general(onehot, xelu, dn, preferred_element_type=jnp.float32)
    sb = lax.dot_general(onehot, root2, dn, preferred_element_type=jnp.float32)
    cnt = lax.dot_general(onehot, jnp.ones((R, 1), jnp.float32), dn,
                          preferred_element_type=jnp.float32)

    @pl.when(i == 0)
    def _():
        oa_ref[...] = sa
        ob_ref[...] = sb
        oc_ref[...] = cnt

    @pl.when(i > 0)
    def _():
        oa_ref[...] += sa
        ob_ref[...] += sb
        oc_ref[...] += cnt

    @pl.when(i == GRID - 1)
    def _():
        cfull = jnp.maximum(oc_ref[...], 1.0)
        oa_ref[...] = oa_ref[...] / cfull
        ob_ref[...] = ob_ref[...] / cfull


_gru_call = pl.pallas_call(
    _gru_body,
    grid=(GRID,),
    in_specs=[
        pl.BlockSpec((L, R, DP), lambda i: (0, i, 0)),
        pl.BlockSpec((R, DP), lambda i: (i, 0)),
        pl.BlockSpec((R, DP), lambda i: (i, 0)),
        pl.BlockSpec((12, DP, DP), lambda i: (0, 0, 0)),
        pl.BlockSpec((8, DP), lambda i: (0, 0)),
        pl.BlockSpec((DP, DP), lambda i: (0, 0)),
        pl.BlockSpec((R, 2), lambda i: (i, 0)),
    ],
    out_specs=[
        pl.BlockSpec((R, DP), lambda i: (i, 0)),
        pl.BlockSpec((R, DP), lambda i: (i, 0)),
    ],
    out_shape=[
        jax.ShapeDtypeStruct((NP, DP), jnp.float32),
        jax.ShapeDtypeStruct((NP, DP), jnp.float32),
    ],
)

_tcb_call = pl.pallas_call(
    _tcb_body,
    grid=(GRID,),
    in_specs=[
        pl.BlockSpec((2, R, DP), lambda i: (0, i, 0)),
        pl.BlockSpec((R, DP), lambda i: (i, 0)),
        pl.BlockSpec((R, 2), lambda i: (i, 0)),
        pl.BlockSpec((128, DP), lambda i: (0, 0)),
        pl.BlockSpec((R, 1), lambda i: (i, 0)),
        pl.BlockSpec((DP, DP), lambda i: (0, 0)),
        pl.BlockSpec((DP, DP), lambda i: (0, 0)),
        pl.BlockSpec((1, DP), lambda i: (0, 0)),
    ],
    out_specs=[
        pl.BlockSpec((R, DP), lambda i: (i, 0)),
        pl.BlockSpec((R, DP), lambda i: (i, 0)),
    ],
    out_shape=[
        jax.ShapeDtypeStruct((NP, DP), jnp.float32),
        jax.ShapeDtypeStruct((NP, DP), jnp.float32),
    ],
)

_tcc_call = pl.pallas_call(
    _tcc_body,
    grid=(GRID,),
    in_specs=[
        pl.BlockSpec((2, R, DP), lambda i: (0, i, 0)),
        pl.BlockSpec((R, DP), lambda i: (i, 0)),
        pl.BlockSpec((R, 2), lambda i: (i, 0)),
        pl.BlockSpec((128, DP), lambda i: (0, 0)),
        pl.BlockSpec((R, 1), lambda i: (i, 0)),
        pl.BlockSpec((1, DP), lambda i: (0, 0)),
    ],
    out_specs=[
        pl.BlockSpec((128, 128), lambda i: (0, 0)),
        pl.BlockSpec((128, 128), lambda i: (0, 0)),
        pl.BlockSpec((128, 1), lambda i: (0, 0)),
    ],
    out_shape=[
        jax.ShapeDtypeStruct((128, 128), jnp.float32),
        jax.ShapeDtypeStruct((128, 128), jnp.float32),
        jax.ShapeDtypeStruct((128, 1), jnp.float32),
    ],
)


def _padw(w):
    return jnp.pad(w, ((0, DP - w.shape[0]), (0, DP - w.shape[1])))


def kernel(merged_tree_feature, merged_tree_edge_index, indices,
           emb, Wih0, Whh0, bih0, bhh0, Wih1, Whh1, bih1, bhh1,
           h0, W1, b1, W2, b2):
    f32 = jnp.float32
    feat3 = jnp.pad(merged_tree_feature.T.astype(jnp.int32),
                    ((0, 0), (0, NP - N))).reshape(NW, GCH, 128)
    src = merged_tree_edge_index[0].astype(jnp.int32)
    dst = merged_tree_edge_index[1].astype(jnp.int32)
    src2 = jnp.pad(src, (0, EP - E),
                   constant_values=NP - 1).reshape(NCH, 128)
    dst2 = jnp.pad(dst, (0, EP - E),
                   constant_values=NP - 1).reshape(NCH, 128)
    idx_p = jnp.pad(indices.astype(jnp.int32), (0, NP - N),
                    constant_values=-1).reshape(NP, 1)

    def gates(Wih, Whh):
        wr, wz, wn = Wih[0:H].T, Wih[H:2 * H].T, Wih[2 * H:3 * H].T
        ur, uz, un = Whh[0:H].T, Whh[H:2 * H].T, Whh[2 * H:3 * H].T
        return [_padw(m) for m in (wr, wz, wn, ur, uz, un)]

    wg = jnp.stack(gates(Wih0, Whh0) + gates(Wih1, Whh1))

    def bvec(v):
        return jnp.pad(v, (0, DP - H))

    bg = jnp.stack([
        bvec(bih0[0:H] + bhh0[0:H]), bvec(bih0[H:2 * H] + bhh0[H:2 * H]),
        bvec(bih0[2 * H:]), bvec(bhh0[2 * H:]),
        bvec(bih1[0:H] + bhh1[0:H]), bvec(bih1[H:2 * H] + bhh1[H:2 * H]),
        bvec(bih1[2 * H:]), bvec(bhh1[2 * H:]),
    ])
    w1_p = _padw(W1)
    w2a = _padw(W2[0:H])
    w2b = _padw(W2[H:2 * H])
    b1_p = bvec(b1).reshape(1, DP)
    b2_p = bvec(b2).reshape(1, DP)

    _emb_gather, _degree, _edge_acc = _sc_kernels()
    emb_p = _pad_call(emb)
    xg = _emb_gather(emb_p, feat3)
    degp = _degree(dst2)
    degt = degp.T

    h00 = jnp.pad(h0[0], ((0, NP - N), (0, DP - H)))
    h01 = jnp.pad(h0[1], ((0, NP - N), (0, DP - H)))
    h_last, xw1s = _gru_call(xg, h00, h01, wg, bg, w1_p, degt)

    acc1 = _edge_acc(xw1s, src2, dst2)

    xw2s, x2 = _tcb_call(acc1, xw1s, degt, h_last, idx_p, w2a, w2b, b1_p)

    acc2 = _edge_acc(xw2s, src2, dst2)

    oa, ob, _ = _tcc_call(acc2, xw2s, degt, x2, idx_p, b2_p)
    return jnp.concatenate([oa[:, 0:H], ob[:, 0:H]], axis=1).astype(f32)


# speedup vs baseline: 1.8139x; 1.4574x over previous
import functools

import jax
import jax.numpy as jnp
from jax import lax
from jax.experimental import pallas as pl
from jax.experimental.pallas import tpu as pltpu
from jax.experimental.pallas import tpu_sc as plsc

N = 10000
NP = 10240
E = 320000
EP = 327680
L = 8
D = 100
DP = 128
H = 100
B = 128
R = 512
GRID = NP // R
NW = 32
TROWS = NP // 16


ECH = EP // NW // 128
ECHS = 16
NCH = EP // 128
FASTC = 0
FCH = 152
SCH = 8
GCH = L * NP // NW // 128


def _emb_gather_body(emb_hbm, idx3_hbm, out_hbm, eidx2, rows0, rows1,
                     gs0, gs1):
    c = lax.axis_index("c")
    s = lax.axis_index("s")
    wid = s * 2 + c
    lrow = wid // 4
    r0 = (wid % 4) * (GCH * 128)
    pltpu.sync_copy(idx3_hbm.at[wid], eidx2)

    pltpu.async_copy(emb_hbm.at[eidx2.at[0]], rows0, gs0)

    def body(j, carry):
        c0 = 2 * j
        c1 = 2 * j + 1
        pltpu.async_copy(emb_hbm.at[eidx2.at[c1]], rows1, gs1)
        pltpu.make_async_copy(emb_hbm.at[eidx2.at[c0]], rows0, gs0).wait()
        pltpu.sync_copy(rows0, out_hbm.at[lrow, pl.ds(r0 + c0 * 128, 128)])

        @pl.when(j < GCH // 2 - 1)
        def _():
            pltpu.async_copy(emb_hbm.at[eidx2.at[c0 + 2]], rows0, gs0)

        pltpu.make_async_copy(emb_hbm.at[eidx2.at[c1]], rows1, gs1).wait()
        pltpu.sync_copy(rows1, out_hbm.at[lrow, pl.ds(r0 + c1 * 128, 128)])
        return carry

    lax.fori_loop(0, GCH // 2, body, 0)


def _degree_body(dst2_hbm, out_hbm, didx2, ones_v, buf_v, acc):
    c = lax.axis_index("c")
    s = lax.axis_index("s")
    wid = s * 2 + c

    for j in range(8):
        ones_v[pl.ds(j * 16, 16)] = jnp.ones((16,), jnp.float32)

    def zrow(i, carry):
        buf_v[pl.ds(i * 16, 16)] = jnp.zeros((16,), jnp.float32)
        return carry

    lax.fori_loop(0, TROWS // 16, zrow, 0)
    pltpu.sync_copy(buf_v, acc.at[pl.ds(s * TROWS, TROWS)])
    plsc.subcore_barrier()

    def outer(o, carry):
        row0 = wid * ECH + o * ECHS
        pltpu.sync_copy(dst2_hbm.at[pl.ds(row0, ECHS)], didx2)

        def body(i, carry2):
            pltpu.sync_copy(ones_v, acc.at[didx2.at[i]], add=True)
            return carry2

        lax.fori_loop(0, ECHS, body, 0)
        return carry

    lax.fori_loop(0, ECH // ECHS, outer, 0)
    plsc.subcore_barrier()
    pltpu.sync_copy(acc.at[pl.ds(s * TROWS, TROWS)], buf_v)
    pltpu.sync_copy(buf_v, out_hbm.at[c, pl.ds(s * TROWS, TROWS)])


def _edge_acc_body(xws_hbm, src2_hbm, dst2_hbm, out_hbm, sidx2, didx2,
                   rows0, rows1, acc, gs0, gs1):
    c = lax.axis_index("c")
    s = lax.axis_index("s")

    def zrow(i, carry):
        for j in range(DP // 16):
            rows0[i, pl.ds(j * 16, 16)] = jnp.zeros((16,), jnp.float32)
        return carry

    lax.fori_loop(0, 128, zrow, 0)

    def zcp(i, carry):
        pltpu.sync_copy(rows0, acc.at[pl.ds(s * TROWS + i * 128, 128)])
        return carry

    lax.fori_loop(0, TROWS // 128, zcp, 0)
    plsc.subcore_barrier()

    nseg = jnp.where(c == FASTC, FCH // ECHS, SCH // ECHS)
    cstart = jnp.where(c == FASTC, s * FCH, 16 * FCH + s * SCH)

    def outer(o, carry):
        row0 = cstart + o * ECHS
        pltpu.sync_copy(src2_hbm.at[pl.ds(row0, ECHS)], sidx2)
        pltpu.sync_copy(dst2_hbm.at[pl.ds(row0, ECHS)], didx2)
        pltpu.async_copy(xws_hbm.at[sidx2.at[0]], rows0, gs0)

        def body(j, carry2):
            c0 = 2 * j
            c1 = 2 * j + 1
            pltpu.async_copy(xws_hbm.at[sidx2.at[c1]], rows1, gs1)
            pltpu.make_async_copy(xws_hbm.at[sidx2.at[c0]], rows0, gs0).wait()
            pltpu.sync_copy(rows0, acc.at[didx2.at[c0]], add=True)

            @pl.when(j < ECHS // 2 - 1)
            def _():
                pltpu.async_copy(xws_hbm.at[sidx2.at[c0 + 2]], rows0, gs0)

            pltpu.make_async_copy(xws_hbm.at[sidx2.at[c1]], rows1, gs1).wait()
            pltpu.sync_copy(rows1, acc.at[didx2.at[c1]], add=True)
            return carry2

        lax.fori_loop(0, ECHS // 2, body, 0)
        return carry

    lax.fori_loop(0, nseg, outer, 0)
    plsc.subcore_barrier()

    def wb(i, carry):
        r0 = s * TROWS + i * 128
        pltpu.sync_copy(acc.at[pl.ds(r0, 128)], rows0)
        pltpu.sync_copy(rows0, out_hbm.at[c, pl.ds(r0, 128)])
        return carry

    lax.fori_loop(0, TROWS // 128, wb, 0)


_SC_CACHE = {}


def _sc_kernels():
    if "emb" not in _SC_CACHE:
        mesh = plsc.VectorSubcoreMesh(core_axis_name="c", subcore_axis_name="s")
        _SC_CACHE["emb"] = pl.kernel(
            _emb_gather_body,
            out_type=jax.ShapeDtypeStruct((L, NP, DP), jnp.float32),
            mesh=mesh,
            scratch_types=[
                pltpu.VMEM((GCH, 128), jnp.int32),
                pltpu.VMEM((128, DP), jnp.float32),
                pltpu.VMEM((128, DP), jnp.float32),
                pltpu.SemaphoreType.DMA,
                pltpu.SemaphoreType.DMA,
            ],
        )
        _SC_CACHE["deg"] = pl.kernel(
            _degree_body,
            out_type=jax.ShapeDtypeStruct((2, NP), jnp.float32),
            mesh=mesh,
            scratch_types=[
                pltpu.VMEM((ECHS, 128), jnp.int32),
                pltpu.VMEM((128,), jnp.float32),
                pltpu.VMEM((TROWS,), jnp.float32),
                pltpu.VMEM_SHARED((NP,), jnp.float32),
            ],
        )
        _SC_CACHE["edge"] = pl.kernel(
            _edge_acc_body,
            out_type=jax.ShapeDtypeStruct((2, NP, DP), jnp.float32),
            mesh=mesh,
            scratch_types=[
                pltpu.VMEM((ECHS, 128), jnp.int32),
                pltpu.VMEM((ECHS, 128), jnp.int32),
                pltpu.VMEM((128, DP), jnp.float32),
                pltpu.VMEM((128, DP), jnp.float32),
                pltpu.VMEM_SHARED((NP, DP), jnp.float32),
                pltpu.SemaphoreType.DMA,
                pltpu.SemaphoreType.DMA,
            ],
        )
    return _SC_CACHE["emb"], _SC_CACHE["deg"], _SC_CACHE["edge"]


VCHUNK = 10000


def _pad_body(emb_ref, out_ref):
    x = emb_ref[...]
    out_ref[...] = jnp.pad(x, ((0, 0), (0, DP - D)))


_pad_call = pl.pallas_call(
    _pad_body,
    grid=(100000 // VCHUNK,),
    in_specs=[pl.BlockSpec((VCHUNK, D), lambda i: (i, 0))],
    out_specs=pl.BlockSpec((VCHUNK, DP), lambda i: (i, 0)),
    out_shape=jax.ShapeDtypeStruct((100000, DP), jnp.float32),
)


def _sig(x):
    return jax.nn.sigmoid(x)


def _elu(x):
    return jnp.where(x > 0, x, jnp.exp(jnp.minimum(x, 0.0)) - 1.0)


def _dot(a, b):
    return jnp.dot(a, b, preferred_element_type=jnp.float32)


def _cell(x, h, wr, wz, wn, ur, uz, un, br, bz, bin_, bhn):
    hb = h.astype(jnp.bfloat16)
    r = _sig(_dot(x, wr) + _dot(hb, ur) + br)
    z = _sig(_dot(x, wz) + _dot(hb, uz) + bz)
    n = jnp.tanh(_dot(x, wn) + bin_ + r * (_dot(hb, un) + bhn))
    return (1.0 - z) * n + z * h


def _gru_body(xg_ref, h00_ref, h01_ref, wg_ref, bg_ref, w1_ref, degt_ref,
              hlast_ref, xw1s_ref):
    h0 = h00_ref[...]
    h1 = h01_ref[...]
    w = [wg_ref[k].astype(jnp.bfloat16) for k in range(12)]
    b = [bg_ref[k:k + 1, :] for k in range(8)]
    for t in range(L):
        x = xg_ref[t].astype(jnp.bfloat16)
        h0 = _cell(x, h0, w[0], w[1], w[2], w[3], w[4], w[5],
                   b[0], b[1], b[2], b[3])
        h1 = _cell(h0.astype(jnp.bfloat16), h1, w[6], w[7], w[8], w[9],
                   w[10], w[11], b[4], b[5], b[6], b[7])
    hlast_ref[...] = h1
    dinv = lax.rsqrt(degt_ref[:, 0:1] + degt_ref[:, 1:2] + 1.0)
    rows = pl.program_id(0) * R + lax.broadcasted_iota(jnp.int32, (R, 1), 0)
    xw1s_ref[...] = jnp.where(rows < N, dinv * jnp.dot(h1, w1_ref[...]), 0.0)


def _tcb_body(acc_ref, xw1s_ref, degt_ref, x1r_ref, idx_ref, w2a_ref, w2b_ref,
              b1_ref, xw2s_ref, x2_ref):
    acc = acc_ref[0] + acc_ref[1]
    dinv = lax.rsqrt(degt_ref[:, 0:1] + degt_ref[:, 1:2] + 1.0)
    x2 = dinv * (acc + xw1s_ref[...]) + b1_ref[...]
    onehot = (idx_ref[...] ==
              lax.broadcasted_iota(jnp.int32, (R, 128), 1)).astype(jnp.float32)
    root = jnp.dot(onehot, x1r_ref[...])
    xw2 = jnp.dot(_elu(x2), w2a_ref[...]) + jnp.dot(_elu(root), w2b_ref[...])
    rows = pl.program_id(0) * R + lax.broadcasted_iota(jnp.int32, (R, 1), 0)
    xw2s_ref[...] = jnp.where(rows < N, dinv * xw2, 0.0)
    x2_ref[...] = x2


def _tcc_body(acc_ref, xw2s_ref, degt_ref, x2r_ref, idx_ref, b2_ref,
              oa_ref, ob_ref, oc_ref):
    i = pl.program_id(0)
    acc = acc_ref[0] + acc_ref[1]
    dinv = lax.rsqrt(degt_ref[:, 0:1] + degt_ref[:, 1:2] + 1.0)
    xelu = _elu(dinv * (acc + xw2s_ref[...]) + b2_ref[...])
    onehot = (idx_ref[...] ==
              lax.broadcasted_iota(jnp.int32, (R, 128), 1)).astype(jnp.float32)
    root2 = jnp.dot(onehot, x2r_ref[...])
    dn = (((0,), (0,)), ((), ()))
    sa = lax.dot_general(onehot, xelu, dn, preferred_element_type=jnp.float32)
    sb = lax.dot_general(onehot, root2, dn, preferred_element_type=jnp.float32)
    cnt = lax.dot_general(onehot, jnp.ones((R, 1), jnp.float32), dn,
                          preferred_element_type=jnp.float32)

    @pl.when(i == 0)
    def _():
        oa_ref[...] = sa
        ob_ref[...] = sb
        oc_ref[...] = cnt

    @pl.when(i > 0)
    def _():
        oa_ref[...] += sa
        ob_ref[...] += sb
        oc_ref[...] += cnt

    @pl.when(i == GRID - 1)
    def _():
        cfull = jnp.maximum(oc_ref[...], 1.0)
        oa_ref[...] = oa_ref[...] / cfull
        ob_ref[...] = ob_ref[...] / cfull


_gru_call = pl.pallas_call(
    _gru_body,
    grid=(GRID,),
    in_specs=[
        pl.BlockSpec((L, R, DP), lambda i: (0, i, 0)),
        pl.BlockSpec((R, DP), lambda i: (i, 0)),
        pl.BlockSpec((R, DP), lambda i: (i, 0)),
        pl.BlockSpec((12, DP, DP), lambda i: (0, 0, 0)),
        pl.BlockSpec((8, DP), lambda i: (0, 0)),
        pl.BlockSpec((DP, DP), lambda i: (0, 0)),
        pl.BlockSpec((R, 2), lambda i: (i, 0)),
    ],
    out_specs=[
        pl.BlockSpec((R, DP), lambda i: (i, 0)),
        pl.BlockSpec((R, DP), lambda i: (i, 0)),
    ],
    out_shape=[
        jax.ShapeDtypeStruct((NP, DP), jnp.float32),
        jax.ShapeDtypeStruct((NP, DP), jnp.float32),
    ],
)

_tcb_call = pl.pallas_call(
    _tcb_body,
    grid=(GRID,),
    in_specs=[
        pl.BlockSpec((2, R, DP), lambda i: (0, i, 0)),
        pl.BlockSpec((R, DP), lambda i: (i, 0)),
        pl.BlockSpec((R, 2), lambda i: (i, 0)),
        pl.BlockSpec((128, DP), lambda i: (0, 0)),
        pl.BlockSpec((R, 1), lambda i: (i, 0)),
        pl.BlockSpec((DP, DP), lambda i: (0, 0)),
        pl.BlockSpec((DP, DP), lambda i: (0, 0)),
        pl.BlockSpec((1, DP), lambda i: (0, 0)),
    ],
    out_specs=[
        pl.BlockSpec((R, DP), lambda i: (i, 0)),
        pl.BlockSpec((R, DP), lambda i: (i, 0)),
    ],
    out_shape=[
        jax.ShapeDtypeStruct((NP, DP), jnp.float32),
        jax.ShapeDtypeStruct((NP, DP), jnp.float32),
    ],
)

_tcc_call = pl.pallas_call(
    _tcc_body,
    grid=(GRID,),
    in_specs=[
        pl.BlockSpec((2, R, DP), lambda i: (0, i, 0)),
        pl.BlockSpec((R, DP), lambda i: (i, 0)),
        pl.BlockSpec((R, 2), lambda i: (i, 0)),
        pl.BlockSpec((128, DP), lambda i: (0, 0)),
        pl.BlockSpec((R, 1), lambda i: (i, 0)),
        pl.BlockSpec((1, DP), lambda i: (0, 0)),
    ],
    out_specs=[
        pl.BlockSpec((128, 128), lambda i: (0, 0)),
        pl.BlockSpec((128, 128), lambda i: (0, 0)),
        pl.BlockSpec((128, 1), lambda i: (0, 0)),
    ],
    out_shape=[
        jax.ShapeDtypeStruct((128, 128), jnp.float32),
        jax.ShapeDtypeStruct((128, 128), jnp.float32),
        jax.ShapeDtypeStruct((128, 1), jnp.float32),
    ],
)


def _padw(w):
    return jnp.pad(w, ((0, DP - w.shape[0]), (0, DP - w.shape[1])))


def kernel(merged_tree_feature, merged_tree_edge_index, indices,
           emb, Wih0, Whh0, bih0, bhh0, Wih1, Whh1, bih1, bhh1,
           h0, W1, b1, W2, b2):
    f32 = jnp.float32
    feat3 = jnp.pad(merged_tree_feature.T.astype(jnp.int32),
                    ((0, 0), (0, NP - N))).reshape(NW, GCH, 128)
    src = merged_tree_edge_index[0].astype(jnp.int32)
    dst = merged_tree_edge_index[1].astype(jnp.int32)
    src2 = jnp.pad(src, (0, EP - E),
                   constant_values=NP - 1).reshape(NCH, 128)
    dst2 = jnp.pad(dst, (0, EP - E),
                   constant_values=NP - 1).reshape(NCH, 128)
    idx_p = jnp.pad(indices.astype(jnp.int32), (0, NP - N),
                    constant_values=-1).reshape(NP, 1)

    def gates(Wih, Whh):
        wr, wz, wn = Wih[0:H].T, Wih[H:2 * H].T, Wih[2 * H:3 * H].T
        ur, uz, un = Whh[0:H].T, Whh[H:2 * H].T, Whh[2 * H:3 * H].T
        return [_padw(m) for m in (wr, wz, wn, ur, uz, un)]

    wg = jnp.stack(gates(Wih0, Whh0) + gates(Wih1, Whh1))

    def bvec(v):
        return jnp.pad(v, (0, DP - H))

    bg = jnp.stack([
        bvec(bih0[0:H] + bhh0[0:H]), bvec(bih0[H:2 * H] + bhh0[H:2 * H]),
        bvec(bih0[2 * H:]), bvec(bhh0[2 * H:]),
        bvec(bih1[0:H] + bhh1[0:H]), bvec(bih1[H:2 * H] + bhh1[H:2 * H]),
        bvec(bih1[2 * H:]), bvec(bhh1[2 * H:]),
    ])
    w1_p = _padw(W1)
    w2a = _padw(W2[0:H])
    w2b = _padw(W2[H:2 * H])
    b1_p = bvec(b1).reshape(1, DP)
    b2_p = bvec(b2).reshape(1, DP)

    _emb_gather, _degree, _edge_acc = _sc_kernels()
    emb_p = _pad_call(emb)
    xg = _emb_gather(emb_p, feat3)
    degp = _degree(dst2)
    degt = degp.T

    h00 = jnp.pad(h0[0], ((0, NP - N), (0, DP - H)))
    h01 = jnp.pad(h0[1], ((0, NP - N), (0, DP - H)))
    h_last, xw1s = _gru_call(xg, h00, h01, wg, bg, w1_p, degt)

    acc1 = _edge_acc(xw1s, src2, dst2)

    xw2s, x2 = _tcb_call(acc1, xw1s, degt, h_last, idx_p, w2a, w2b, b1_p)

    acc2 = _edge_acc(xw2s, src2, dst2)

    oa, ob, _ = _tcc_call(acc2, xw2s, degt, x2, idx_p, b2_p)
    return jnp.concatenate([oa[:, 0:H], ob[:, 0:H]], axis=1).astype(f32)
